# baseline (speedup 1.0000x reference)
import functools

import jax
import jax.numpy as jnp
from jax import lax
from jax.experimental import pallas as pl
from jax.experimental.pallas import tpu as pltpu
from jax.experimental.pallas import tpu_sc as plsc

EMBED = 768
ROWS = 4 * 8192
NC = 2
NS = 16
NW = NC * NS
ROWS_PER_W = ROWS // NW
REP = 128
CHUNKS = ROWS_PER_W // REP
CHUNK_WORDS = REP * EMBED
VREGS_PER_ROW = EMBED // 16


def _sc_fill(tok_hbm, out_hbm, tok_v, buf_v, sem):
    wid = lax.axis_index("s") * NC + lax.axis_index("c")
    pltpu.sync_copy(tok_hbm, tok_v)

    def rep_body(r, carry):
        for j in range(VREGS_PER_ROW):
            buf_v[pl.ds(r * EMBED + j * 16, 16)] = tok_v[pl.ds(j * 16, 16)]
        return carry

    lax.fori_loop(0, REP, rep_body, 0)

    base = wid * ROWS_PER_W * EMBED
    copies = [
        pltpu.async_copy(
            buf_v, out_hbm.at[pl.ds(base + c * CHUNK_WORDS, CHUNK_WORDS)], sem
        )
        for c in range(CHUNKS)
    ]
    for cp in copies:
        cp.wait()


def kernel(x, mask_token):
    del x
    tok = mask_token.reshape(EMBED)
    mesh = plsc.VectorSubcoreMesh(core_axis_name="c", subcore_axis_name="s")
    fill = functools.partial(
        pl.kernel,
        mesh=mesh,
        out_type=jax.ShapeDtypeStruct((ROWS * EMBED,), jnp.float32),
        scratch_types=[
            pltpu.VMEM((EMBED,), jnp.float32),
            pltpu.VMEM((REP * EMBED,), jnp.float32),
            pltpu.SemaphoreType.DMA,
        ],
    )(_sc_fill)
    out = fill(tok)
    return out.reshape(4, 8192, EMBED)


# speedup vs baseline: 1.1193x; 1.1193x over previous
import functools

import jax
import jax.numpy as jnp
from jax import lax
from jax.experimental import pallas as pl
from jax.experimental.pallas import tpu as pltpu
from jax.experimental.pallas import tpu_sc as plsc

EMBED = 768
ROWS = 4 * 8192
NC = 2
NS = 16
NW = NC * NS
ROWS_PER_W = ROWS // NW
REP = 64
CHUNKS = ROWS_PER_W // REP
CHUNK_WORDS = REP * EMBED
VREGS_PER_ROW = EMBED // 16


def _sc_fill(tok_hbm, out_hbm, tok_v, buf_v, sem):
    wid = lax.axis_index("s") * NC + lax.axis_index("c")
    pltpu.sync_copy(tok_hbm, tok_v)

    row = [tok_v[pl.ds(j * 16, 16)] for j in range(VREGS_PER_ROW)]

    def rep_body(r, carry):
        for j in range(VREGS_PER_ROW):
            buf_v[pl.ds(r * EMBED + j * 16, 16)] = row[j]
        return carry

    lax.fori_loop(0, REP, rep_body, 0)

    base = wid * ROWS_PER_W * EMBED
    copies = [
        pltpu.async_copy(
            buf_v, out_hbm.at[pl.ds(base + c * CHUNK_WORDS, CHUNK_WORDS)], sem
        )
        for c in range(CHUNKS)
    ]
    for cp in copies:
        cp.wait()


def kernel(x, mask_token):
    del x
    tok = mask_token.reshape(EMBED)
    mesh = plsc.VectorSubcoreMesh(core_axis_name="c", subcore_axis_name="s")
    fill = functools.partial(
        pl.kernel,
        mesh=mesh,
        out_type=jax.ShapeDtypeStruct((ROWS * EMBED,), jnp.float32),
        scratch_types=[
            pltpu.VMEM((EMBED,), jnp.float32),
            pltpu.VMEM((REP * EMBED,), jnp.float32),
            pltpu.SemaphoreType.DMA,
        ],
    )(_sc_fill)
    out = fill(tok)
    return out.reshape(4, 8192, EMBED)
